# baseline (speedup 1.0000x reference)
import functools

import jax
import jax.numpy as jnp
from jax import lax
from jax.experimental import pallas as pl
from jax.experimental.pallas import tpu as pltpu
from jax.experimental.pallas import tpu_sc as plsc

N = 50000
E = 800000
CTX = 128
TGT = 32
H = 64
G = 64
FUSION = H + H // 2

NC = 2
NS = 16
K = 128
N_CHUNKS = 392
EPT = N_CHUNKS * K
E_PAD = EPT * NS
Q = 112
N_CYC = EPT // (2 * Q)
RPT = 3128
RPT_LAST = N - 15 * RPT
PAD_DST = N
DEG_PAD = N + 48
DRPT = DEG_PAD // NS

_mesh = plsc.VectorSubcoreMesh(core_axis_name="c", subcore_axis_name="s")


def _deg_body(dstA, dstT, zf, onesf, degA, degT, idx2, ones_v, shared, sem):
    c = lax.axis_index("c")
    s = lax.axis_index("s")
    pltpu.sync_copy(onesf, ones_v)

    def work(dst_ref, out_ref):
        r0 = pl.multiple_of(s * DRPT, 8)
        pltpu.sync_copy(zf, shared.at[pl.ds(r0, DRPT)])

        c0 = pl.multiple_of(s * EPT, 8)
        pltpu.sync_copy(dst_ref.at[pl.ds(c0, EPT)], idx2)
        plsc.subcore_barrier()
        pltpu.sync_copy(ones_v, shared.at[idx2], add=True)
        plsc.subcore_barrier()
        pltpu.sync_copy(shared.at[pl.ds(r0, DRPT)], out_ref.at[pl.ds(r0, DRPT)])

    @pl.when(c == 0)
    def _():
        work(dstA, degA)

    @pl.when(c == 1)
    def _():
        work(dstT, degT)


def _sc_degrees(dstA_p, dstT_p):
    zf = jnp.zeros((DRPT,), jnp.float32)
    onesf = jnp.ones((EPT,), jnp.float32)
    return pl.kernel(
        _deg_body,
        out_type=(jax.ShapeDtypeStruct((DEG_PAD,), jnp.float32),
                  jax.ShapeDtypeStruct((DEG_PAD,), jnp.float32)),
        mesh=_mesh,
        scratch_types=[
            pltpu.VMEM((EPT,), jnp.int32),
            pltpu.VMEM((EPT,), jnp.float32),
            pltpu.VMEM_SHARED((DEG_PAD,), jnp.float32),
            pltpu.SemaphoreType.DMA,
        ],
        compiler_params=pltpu.CompilerParams(use_tc_tiling_on_sc=False),
    )(dstA_p, dstT_p, zf, onesf)


def _edge_body(h0, h1, src, dst, out0, out1, idx_s0, idx_d0, idx_s1, idx_d1,
               rows, shared, isem, gsem, ssem):
    c = lax.axis_index("c")
    s = lax.axis_index("s")

    def work(h_ref, out_ref):
        r0 = pl.multiple_of(s * RPT, 8)

        @pl.when(s < NS - 1)
        def _():
            pltpu.sync_copy(h_ref.at[pl.ds(r0, RPT)], shared.at[pl.ds(r0, RPT)])

        @pl.when(s == NS - 1)
        def _():
            pltpu.sync_copy(h_ref.at[pl.ds(15 * RPT, RPT_LAST)],
                            shared.at[pl.ds(15 * RPT, RPT_LAST)])

        plsc.subcore_barrier()

        def cyc(u, carry):
            off0 = pl.multiple_of(s * EPT + u * 2 * Q, 8)
            off1 = pl.multiple_of(s * EPT + u * 2 * Q + Q, 8)
            d0 = pltpu.async_copy(src.at[pl.ds(off0, Q)], idx_s0, isem)
            d1 = pltpu.async_copy(dst.at[pl.ds(off0, Q)], idx_d0, isem)
            d2 = pltpu.async_copy(src.at[pl.ds(off1, Q)], idx_s1, isem)
            d3 = pltpu.async_copy(dst.at[pl.ds(off1, Q)], idx_d1, isem)
            d0.wait(); d1.wait(); d2.wait(); d3.wait()
            pltpu.async_copy(h_ref.at[idx_s0], rows.at[0], gsem).wait()
            s0 = pltpu.async_copy(rows.at[0], shared.at[idx_d0], ssem,
                                  add=True)
            g1 = pltpu.async_copy(h_ref.at[idx_s1], rows.at[1], gsem)
            g1.wait()
            s0.wait()
            pltpu.async_copy(rows.at[1], shared.at[idx_d1], ssem,
                             add=True).wait()
            return carry

        lax.fori_loop(0, N_CYC, cyc, 0)
        plsc.subcore_barrier()

        @pl.when(s < NS - 1)
        def _():
            pltpu.sync_copy(shared.at[pl.ds(r0, RPT)], out_ref.at[pl.ds(r0, RPT)])

        @pl.when(s == NS - 1)
        def _():
            pltpu.sync_copy(shared.at[pl.ds(15 * RPT, RPT_LAST)],
                            out_ref.at[pl.ds(15 * RPT, RPT_LAST)])

    @pl.when(c == 0)
    def _():
        work(h0, out0)

    @pl.when(c == 1)
    def _():
        work(h1, out1)


def _sc_edge_pass(h0, h1, src_p, dst_p):
    return pl.kernel(
        _edge_body,
        out_type=(jax.ShapeDtypeStruct((N, G // 2), jnp.float32),
                  jax.ShapeDtypeStruct((N, G // 2), jnp.float32)),
        mesh=_mesh,
        scratch_types=[
            pltpu.VMEM((Q,), jnp.int32),
            pltpu.VMEM((Q,), jnp.int32),
            pltpu.VMEM((Q,), jnp.int32),
            pltpu.VMEM((Q,), jnp.int32),
            pltpu.VMEM((2, Q, G // 2), jnp.float32),
            pltpu.VMEM_SHARED((N + 8, G // 2), jnp.float32),
            pltpu.SemaphoreType.DMA,
            pltpu.SemaphoreType.DMA,
            pltpu.SemaphoreType.DMA,
        ],
        compiler_params=pltpu.CompilerParams(use_tc_tiling_on_sc=False),
    )(h0, h1, src_p, dst_p)


_B = 2000
_GRID = N // _B


def _relu(x):
    return jnp.maximum(x, 0.0)


def _dot(a, b):
    return jnp.dot(a, b, preferred_element_type=jnp.float32)


def _enc_body(ctx, tlog, mf, degA, degT,
              W1, b1, g1, be1, m1, v1, W2, b2, Wt, bt, mtok, gW_a, gW_t,
              fused_o, hA0, hA1, hT0, hT1):
    h = _dot(ctx[...], W1[...]) + b1[...]
    h = _relu((h - m1[...]) * lax.rsqrt(v1[...] + 1e-5) * g1[...] + be1[...])
    ctx_emb = _relu(_dot(h, W2[...]) + b2[...])
    mfv = mf[...]
    mt = tlog[...] * (1.0 - mfv) + mtok[...] * mfv
    tgt_emb = _relu(_dot(mt, Wt[...]) + bt[...])
    fused = jnp.concatenate([ctx_emb, tgt_emb], axis=-1)
    fused_o[...] = fused
    dinvA = lax.rsqrt(degA[...] + 1.0)
    dinvT = lax.rsqrt(degT[...] + 1.0)
    ha = dinvA * _dot(fused, gW_a[...])
    ht = dinvT * _dot(fused, gW_t[...])
    hA0[...] = ha[:, :G // 2]
    hA1[...] = ha[:, G // 2:]
    hT0[...] = ht[:, :G // 2]
    hT1[...] = ht[:, G // 2:]


def _mid_body(aA0, aA1, aT0, aT1, degA, degT, b_a, b_t, W2a, W2t,
              hA0, hA1, hT0, hT1):
    dinvA = lax.rsqrt(degA[...] + 1.0)
    dinvT = lax.rsqrt(degT[...] + 1.0)
    accA = jnp.concatenate([aA0[...], aA1[...]], axis=-1)
    accT = jnp.concatenate([aT0[...], aT1[...]], axis=-1)
    hs = _relu(dinvA * accA + b_a[...])
    ht = _relu(dinvT * accT + b_t[...])
    t2 = dinvA * _dot(hs, W2a[...])
    u2 = dinvT * _dot(ht, W2t[...])
    hA0[...] = t2[:, :G // 2]
    hA1[...] = t2[:, G // 2:]
    hT0[...] = u2[:, :G // 2]
    hT1[...] = u2[:, G // 2:]


def _fin_body(aA0, aA1, aT0, aT1, degA, degT, fused, b_a, b_t,
              Wp1, bp1, g2, be2, m2, v2, Wp2, bp2, Wp3, bp3, out):
    dinvA = lax.rsqrt(degA[...] + 1.0)
    dinvT = lax.rsqrt(degT[...] + 1.0)
    hs2 = dinvA * jnp.concatenate([aA0[...], aA1[...]], axis=-1) + b_a[...]
    ht2 = dinvT * jnp.concatenate([aT0[...], aT1[...]], axis=-1) + b_t[...]
    combined = jnp.concatenate([hs2 + ht2, fused[...]], axis=-1)
    o = _dot(combined, Wp1[...]) + bp1[...]
    o = _relu((o - m2[...]) * lax.rsqrt(v2[...] + 1e-5) * g2[...] + be2[...])
    o = _relu(_dot(o, Wp2[...]) + bp2[...])
    out[...] = _dot(o, Wp3[...]) + bp3[...]


def _row_spec(w):
    return pl.BlockSpec((_B, w), lambda i: (i, 0))


def _full_spec(a, b):
    return pl.BlockSpec((a, b), lambda i: (0, 0))


def _tc_call(body, in_specs, out_specs, out_shapes):
    return pl.pallas_call(
        body,
        grid=(_GRID,),
        in_specs=in_specs,
        out_specs=out_specs,
        out_shape=out_shapes,
        compiler_params=pltpu.CompilerParams(
            dimension_semantics=("arbitrary",)),
    )


def kernel(context, target_log, mask, adj_edge_index, transit_edge_index,
           W1, b1, bn1_g, bn1_b, bn1_m, bn1_v, W2, b2, Wt, bt, mask_token,
           gcn1_W, gcn1_b, gcn2_W, gcn2_b, tg1_W, tg1_b, tg2_W, tg2_b,
           graph_alpha, Wp1, bp1, bn2_g, bn2_b, bn2_m, bn2_v, Wp2, bp2,
           Wp3, bp3):
    mf = mask.astype(jnp.float32)
    pad_src = jnp.zeros((E_PAD - E,), jnp.int32)
    pad_dst = jnp.full((E_PAD - E,), PAD_DST, jnp.int32)
    srcA = jnp.concatenate([adj_edge_index[0], pad_src])
    dstA = jnp.concatenate([adj_edge_index[1], pad_dst])
    srcT = jnp.concatenate([transit_edge_index[0], pad_src])
    dstT = jnp.concatenate([transit_edge_index[1], pad_dst])
    alpha = jax.nn.sigmoid(graph_alpha)
    gcn2_Ws = gcn2_W * alpha
    gcn2_bs = (gcn2_b * alpha).reshape(1, G)
    tg2_Ws = tg2_W * (1.0 - alpha)
    tg2_bs = (tg2_b * (1.0 - alpha)).reshape(1, G)
    row = lambda v: v.reshape(1, -1)

    degA, degT = _sc_degrees(dstA, dstT)
    degA = degA[:N].reshape(N, 1)
    degT = degT[:N].reshape(N, 1)

    enc = _tc_call(
        _enc_body,
        in_specs=[
            _row_spec(CTX), _row_spec(TGT), _row_spec(TGT),
            _row_spec(1), _row_spec(1),
            _full_spec(CTX, H), _full_spec(1, H),
            _full_spec(1, H), _full_spec(1, H), _full_spec(1, H), _full_spec(1, H),
            _full_spec(H, H), _full_spec(1, H),
            _full_spec(TGT, H // 2), _full_spec(1, H // 2), _full_spec(1, TGT),
            _full_spec(FUSION, G), _full_spec(FUSION, G),
        ],
        out_specs=[_row_spec(FUSION)] + [_row_spec(G // 2)] * 4,
        out_shapes=[jax.ShapeDtypeStruct((N, FUSION), jnp.float32)] +
                   [jax.ShapeDtypeStruct((N, G // 2), jnp.float32)] * 4,
    )
    fused, hA0, hA1, hT0, hT1 = enc(
        context, target_log, mf, degA, degT,
        W1, row(b1), row(bn1_g), row(bn1_b), row(bn1_m), row(bn1_v),
        W2, row(b2), Wt, row(bt), mask_token, gcn1_W, tg1_W)

    aA0, aA1 = _sc_edge_pass(hA0, hA1, srcA, dstA)
    aT0, aT1 = _sc_edge_pass(hT0, hT1, srcT, dstT)

    mid = _tc_call(
        _mid_body,
        in_specs=[_row_spec(G // 2)] * 4 + [_row_spec(1)] * 2 +
                 [_full_spec(1, G)] * 2 + [_full_spec(G, G)] * 2,
        out_specs=[_row_spec(G // 2)] * 4,
        out_shapes=[jax.ShapeDtypeStruct((N, G // 2), jnp.float32)] * 4,
    )
    hA20, hA21, hT20, hT21 = mid(
        aA0, aA1, aT0, aT1, degA, degT,
        row(gcn1_b), row(tg1_b), gcn2_Ws, tg2_Ws)

    bA0, bA1 = _sc_edge_pass(hA20, hA21, srcA, dstA)
    bT0, bT1 = _sc_edge_pass(hT20, hT21, srcT, dstT)

    fin = _tc_call(
        _fin_body,
        in_specs=[_row_spec(G // 2)] * 4 + [_row_spec(1)] * 2 +
                 [_row_spec(FUSION)] + [_full_spec(1, G)] * 2 +
                 [_full_spec(G + FUSION, H), _full_spec(1, H)] +
                 [_full_spec(1, H)] * 4 +
                 [_full_spec(H, H // 2), _full_spec(1, H // 2),
                  _full_spec(H // 2, TGT), _full_spec(1, TGT)],
        out_specs=[_row_spec(TGT)],
        out_shapes=[jax.ShapeDtypeStruct((N, TGT), jnp.float32)],
    )
    (out,) = fin(
        bA0, bA1, bT0, bT1, degA, degT, fused,
        gcn2_bs, tg2_bs, Wp1, row(bp1), row(bn2_g), row(bn2_b),
        row(bn2_m), row(bn2_v), Wp2, row(bp2), Wp3, row(bp3))
    return out


# speedup vs baseline: 1.0205x; 1.0205x over previous
import functools

import jax
import jax.numpy as jnp
from jax import lax
from jax.experimental import pallas as pl
from jax.experimental.pallas import tpu as pltpu
from jax.experimental.pallas import tpu_sc as plsc

N = 50000
E = 800000
CTX = 128
TGT = 32
H = 64
G = 64
FUSION = H + H // 2

NC = 2
NS = 16
K = 128
N_CHUNKS = 392
EPT = N_CHUNKS * K
E_PAD = EPT * NS
Q = 224
N_GRP = EPT // Q
RPT = 3128
RPT_LAST = N - 15 * RPT
PAD_DST = N
DEG_PAD = N + 48
DRPT = DEG_PAD // NS

_mesh = plsc.VectorSubcoreMesh(core_axis_name="c", subcore_axis_name="s")
_sc_params = pltpu.CompilerParams(use_tc_tiling_on_sc=False)


def _deg_body(dstA, dstT, zf, onesf, degA, degT, idx2, ones_v, shared, sem):
    c = lax.axis_index("c")
    s = lax.axis_index("s")
    pltpu.sync_copy(onesf, ones_v)

    def work(dst_ref, out_ref):
        r0 = pl.multiple_of(s * DRPT, 8)
        pltpu.sync_copy(zf, shared.at[pl.ds(r0, DRPT)])
        c0 = pl.multiple_of(s * EPT, 8)
        pltpu.sync_copy(dst_ref.at[pl.ds(c0, EPT)], idx2)
        plsc.subcore_barrier()
        pltpu.sync_copy(ones_v, shared.at[idx2], add=True)
        plsc.subcore_barrier()
        pltpu.sync_copy(shared.at[pl.ds(r0, DRPT)], out_ref.at[pl.ds(r0, DRPT)])

    @pl.when(c == 0)
    def _():
        work(dstA, degA)

    @pl.when(c == 1)
    def _():
        work(dstT, degT)


def _sc_degrees(dstA_p, dstT_p):
    zf = jnp.zeros((DRPT,), jnp.float32)
    onesf = jnp.ones((EPT,), jnp.float32)
    return pl.kernel(
        _deg_body,
        out_type=(jax.ShapeDtypeStruct((DEG_PAD,), jnp.float32),
                  jax.ShapeDtypeStruct((DEG_PAD,), jnp.float32)),
        mesh=_mesh,
        scratch_types=[
            pltpu.VMEM((EPT,), jnp.int32),
            pltpu.VMEM((EPT,), jnp.float32),
            pltpu.VMEM_SHARED((DEG_PAD,), jnp.float32),
            pltpu.SemaphoreType.DMA,
        ],
        compiler_params=_sc_params,
    )(dstA_p, dstT_p, zf, onesf)


def _layer_body(h0, h1, h2, h3, srcA, dstA, srcT, dstT, out,
                idx_s, idx_d, rows, shared, gsem):
    c = lax.axis_index("c")
    s = lax.axis_index("s")

    def halfpass(h_ref, src, dst, co):
        r0 = pl.multiple_of(s * RPT, 8)

        @pl.when(s < NS - 1)
        def _():
            pltpu.sync_copy(h_ref.at[pl.ds(r0, RPT)], shared.at[pl.ds(r0, RPT)])

        @pl.when(s == NS - 1)
        def _():
            pltpu.sync_copy(h_ref.at[pl.ds(15 * RPT, RPT_LAST)],
                            shared.at[pl.ds(15 * RPT, RPT_LAST)])

        plsc.subcore_barrier()

        def grp(u, carry):
            off = pl.multiple_of(s * EPT + u * Q, 8)
            pltpu.sync_copy(src.at[pl.ds(off, Q)], idx_s)
            pltpu.sync_copy(dst.at[pl.ds(off, Q)], idx_d)
            pltpu.async_copy(h_ref.at[idx_s], rows, gsem).wait()
            pltpu.sync_copy(rows, shared.at[idx_d], add=True)
            return carry

        lax.fori_loop(0, N_GRP, grp, 0)
        plsc.subcore_barrier()

        @pl.when(s < NS - 1)
        def _():
            pltpu.sync_copy(shared.at[pl.ds(r0, RPT)],
                            out.at[pl.ds(r0, RPT), pl.ds(co, G // 2)])

        @pl.when(s == NS - 1)
        def _():
            pltpu.sync_copy(shared.at[pl.ds(15 * RPT, RPT_LAST)],
                            out.at[pl.ds(15 * RPT, RPT_LAST), pl.ds(co, G // 2)])

        plsc.subcore_barrier()

    @pl.when(c == 0)
    def _():
        halfpass(h0, srcA, dstA, 0)
        halfpass(h2, srcT, dstT, G)

    @pl.when(c == 1)
    def _():
        halfpass(h1, srcA, dstA, G // 2)
        halfpass(h3, srcT, dstT, G + G // 2)


def _sc_layer(h0, h1, h2, h3, srcA, dstA, srcT, dstT):
    return pl.kernel(
        _layer_body,
        out_type=jax.ShapeDtypeStruct((N, 2 * G), jnp.float32),
        mesh=_mesh,
        scratch_types=[
            pltpu.VMEM((Q,), jnp.int32),
            pltpu.VMEM((Q,), jnp.int32),
            pltpu.VMEM((Q, G // 2), jnp.float32),
            pltpu.VMEM_SHARED((N + 8, G // 2), jnp.float32),
            pltpu.SemaphoreType.DMA,
        ],
        compiler_params=_sc_params,
    )(h0, h1, h2, h3, srcA, dstA, srcT, dstT)


_B = 2000
_GRID = N // _B


def _relu(x):
    return jnp.maximum(x, 0.0)


def _dot(a, b):
    return jnp.dot(a, b, preferred_element_type=jnp.float32)


def _enc_body(ctx, tlog, mf, degA, degT,
              W1, b1, g1, be1, m1, v1, W2, b2, Wt, bt, mtok, gW_a, gW_t,
              fused_o, hA0, hA1, hT0, hT1):
    h = _dot(ctx[...], W1[...]) + b1[...]
    h = _relu((h - m1[...]) * lax.rsqrt(v1[...] + 1e-5) * g1[...] + be1[...])
    ctx_emb = _relu(_dot(h, W2[...]) + b2[...])
    mfv = mf[...]
    mt = tlog[...] * (1.0 - mfv) + mtok[...] * mfv
    tgt_emb = _relu(_dot(mt, Wt[...]) + bt[...])
    fused = jnp.concatenate([ctx_emb, tgt_emb], axis=-1)
    fused_o[...] = fused
    dinvA = lax.rsqrt(degA[...] + 1.0)
    dinvT = lax.rsqrt(degT[...] + 1.0)
    ha = dinvA * _dot(fused, gW_a[...])
    ht = dinvT * _dot(fused, gW_t[...])
    hA0[...] = ha[:, :G // 2]
    hA1[...] = ha[:, G // 2:]
    hT0[...] = ht[:, :G // 2]
    hT1[...] = ht[:, G // 2:]


def _mid_body(acc, degA, degT, b_a, b_t, W2a, W2t, hA0, hA1, hT0, hT1):
    dinvA = lax.rsqrt(degA[...] + 1.0)
    dinvT = lax.rsqrt(degT[...] + 1.0)
    accv = acc[...]
    hs = _relu(dinvA * accv[:, :G] + b_a[...])
    ht = _relu(dinvT * accv[:, G:] + b_t[...])
    t2 = dinvA * _dot(hs, W2a[...])
    u2 = dinvT * _dot(ht, W2t[...])
    hA0[...] = t2[:, :G // 2]
    hA1[...] = t2[:, G // 2:]
    hT0[...] = u2[:, :G // 2]
    hT1[...] = u2[:, G // 2:]


def _fin_body(acc, degA, degT, fused, b_a, b_t,
              Wp1, bp1, g2, be2, m2, v2, Wp2, bp2, Wp3, bp3, out):
    dinvA = lax.rsqrt(degA[...] + 1.0)
    dinvT = lax.rsqrt(degT[...] + 1.0)
    accv = acc[...]
    hs2 = dinvA * accv[:, :G] + b_a[...]
    ht2 = dinvT * accv[:, G:] + b_t[...]
    combined = jnp.concatenate([hs2 + ht2, fused[...]], axis=-1)
    o = _dot(combined, Wp1[...]) + bp1[...]
    o = _relu((o - m2[...]) * lax.rsqrt(v2[...] + 1e-5) * g2[...] + be2[...])
    o = _relu(_dot(o, Wp2[...]) + bp2[...])
    out[...] = _dot(o, Wp3[...]) + bp3[...]


def _row_spec(w):
    return pl.BlockSpec((_B, w), lambda i: (i, 0))


def _full_spec(a, b):
    return pl.BlockSpec((a, b), lambda i: (0, 0))


def _tc_call(body, in_specs, out_specs, out_shapes):
    return pl.pallas_call(
        body,
        grid=(_GRID,),
        in_specs=in_specs,
        out_specs=out_specs,
        out_shape=out_shapes,
        compiler_params=pltpu.CompilerParams(
            dimension_semantics=("arbitrary",)),
    )


def kernel(context, target_log, mask, adj_edge_index, transit_edge_index,
           W1, b1, bn1_g, bn1_b, bn1_m, bn1_v, W2, b2, Wt, bt, mask_token,
           gcn1_W, gcn1_b, gcn2_W, gcn2_b, tg1_W, tg1_b, tg2_W, tg2_b,
           graph_alpha, Wp1, bp1, bn2_g, bn2_b, bn2_m, bn2_v, Wp2, bp2,
           Wp3, bp3):
    mf = mask.astype(jnp.float32)
    pad_src = jnp.zeros((E_PAD - E,), jnp.int32)
    pad_dst = jnp.full((E_PAD - E,), PAD_DST, jnp.int32)
    srcA = jnp.concatenate([adj_edge_index[0], pad_src])
    dstA = jnp.concatenate([adj_edge_index[1], pad_dst])
    srcT = jnp.concatenate([transit_edge_index[0], pad_src])
    dstT = jnp.concatenate([transit_edge_index[1], pad_dst])
    alpha = jax.nn.sigmoid(graph_alpha)
    gcn2_Ws = gcn2_W * alpha
    gcn2_bs = (gcn2_b * alpha).reshape(1, G)
    tg2_Ws = tg2_W * (1.0 - alpha)
    tg2_bs = (tg2_b * (1.0 - alpha)).reshape(1, G)
    row = lambda v: v.reshape(1, -1)

    degA, degT = _sc_degrees(dstA, dstT)
    degA = degA[:N].reshape(N, 1)
    degT = degT[:N].reshape(N, 1)

    enc = _tc_call(
        _enc_body,
        in_specs=[
            _row_spec(CTX), _row_spec(TGT), _row_spec(TGT),
            _row_spec(1), _row_spec(1),
            _full_spec(CTX, H), _full_spec(1, H),
            _full_spec(1, H), _full_spec(1, H), _full_spec(1, H), _full_spec(1, H),
            _full_spec(H, H), _full_spec(1, H),
            _full_spec(TGT, H // 2), _full_spec(1, H // 2), _full_spec(1, TGT),
            _full_spec(FUSION, G), _full_spec(FUSION, G),
        ],
        out_specs=[_row_spec(FUSION)] + [_row_spec(G // 2)] * 4,
        out_shapes=[jax.ShapeDtypeStruct((N, FUSION), jnp.float32)] +
                   [jax.ShapeDtypeStruct((N, G // 2), jnp.float32)] * 4,
    )
    fused, hA0, hA1, hT0, hT1 = enc(
        context, target_log, mf, degA, degT,
        W1, row(b1), row(bn1_g), row(bn1_b), row(bn1_m), row(bn1_v),
        W2, row(b2), Wt, row(bt), mask_token, gcn1_W, tg1_W)

    acc1 = _sc_layer(hA0, hA1, hT0, hT1, srcA, dstA, srcT, dstT)

    mid = _tc_call(
        _mid_body,
        in_specs=[_row_spec(2 * G)] + [_row_spec(1)] * 2 +
                 [_full_spec(1, G)] * 2 + [_full_spec(G, G)] * 2,
        out_specs=[_row_spec(G // 2)] * 4,
        out_shapes=[jax.ShapeDtypeStruct((N, G // 2), jnp.float32)] * 4,
    )
    hA20, hA21, hT20, hT21 = mid(
        acc1, degA, degT, row(gcn1_b), row(tg1_b), gcn2_Ws, tg2_Ws)

    acc2 = _sc_layer(hA20, hA21, hT20, hT21, srcA, dstA, srcT, dstT)

    fin = _tc_call(
        _fin_body,
        in_specs=[_row_spec(2 * G)] + [_row_spec(1)] * 2 +
                 [_row_spec(FUSION)] + [_full_spec(1, G)] * 2 +
                 [_full_spec(G + FUSION, H), _full_spec(1, H)] +
                 [_full_spec(1, H)] * 4 +
                 [_full_spec(H, H // 2), _full_spec(1, H // 2),
                  _full_spec(H // 2, TGT), _full_spec(1, TGT)],
        out_specs=[_row_spec(TGT)],
        out_shapes=[jax.ShapeDtypeStruct((N, TGT), jnp.float32)],
    )
    (out,) = fin(
        acc2, degA, degT, fused,
        gcn2_bs, tg2_bs, Wp1, row(bp1), row(bn2_g), row(bn2_b),
        row(bn2_m), row(bn2_v), Wp2, row(bp2), Wp3, row(bp3))
    return out


# speedup vs baseline: 1.0239x; 1.0033x over previous
import functools

import jax
import jax.numpy as jnp
from jax import lax
from jax.experimental import pallas as pl
from jax.experimental.pallas import tpu as pltpu
from jax.experimental.pallas import tpu_sc as plsc

N = 50000
E = 800000
CTX = 128
TGT = 32
H = 64
G = 64
FUSION = H + H // 2

NC = 2
NS = 16
K = 128
N_CHUNKS = 392
EPT = N_CHUNKS * K
E_PAD = EPT * NS
Q = 224
N_GRP = EPT // Q
RPT = 3128
RPT_LAST = N - 15 * RPT
PAD_DST = N
DEG_PAD = N + 48
DRPT = DEG_PAD // NS

_mesh = plsc.VectorSubcoreMesh(core_axis_name="c", subcore_axis_name="s")
_sc_params = pltpu.CompilerParams(use_tc_tiling_on_sc=False)


def _deg_body(dstA, dstT, zf, onesf, degA, degT, idx2, ones_v, shared, sem):
    c = lax.axis_index("c")
    s = lax.axis_index("s")
    pltpu.sync_copy(onesf, ones_v)

    def work(dst_ref, out_ref):
        r0 = pl.multiple_of(s * DRPT, 8)
        pltpu.sync_copy(zf, shared.at[pl.ds(r0, DRPT)])
        c0 = pl.multiple_of(s * EPT, 8)
        pltpu.sync_copy(dst_ref.at[pl.ds(c0, EPT)], idx2)
        plsc.subcore_barrier()
        pltpu.sync_copy(ones_v, shared.at[idx2], add=True)
        plsc.subcore_barrier()
        pltpu.sync_copy(shared.at[pl.ds(r0, DRPT)], out_ref.at[pl.ds(r0, DRPT)])

    @pl.when(c == 0)
    def _():
        work(dstA, degA)

    @pl.when(c == 1)
    def _():
        work(dstT, degT)


def _sc_degrees(dstA_p, dstT_p):
    zf = jnp.zeros((DRPT,), jnp.float32)
    onesf = jnp.ones((EPT,), jnp.float32)
    return pl.kernel(
        _deg_body,
        out_type=(jax.ShapeDtypeStruct((DEG_PAD,), jnp.float32),
                  jax.ShapeDtypeStruct((DEG_PAD,), jnp.float32)),
        mesh=_mesh,
        scratch_types=[
            pltpu.VMEM((EPT,), jnp.int32),
            pltpu.VMEM((EPT,), jnp.float32),
            pltpu.VMEM_SHARED((DEG_PAD,), jnp.float32),
            pltpu.SemaphoreType.DMA,
        ],
        compiler_params=_sc_params,
    )(dstA_p, dstT_p, zf, onesf)


def _layer_body(h0, h1, h2, h3, srcA, dstA, srcT, dstT, out,
                idx_s, idx_d, rows, shared, gsem):
    c = lax.axis_index("c")
    s = lax.axis_index("s")

    def halfpass(h_ref, src, dst, co):
        r0 = pl.multiple_of(s * RPT, 8)

        @pl.when(s < NS - 1)
        def _():
            pltpu.sync_copy(h_ref.at[pl.ds(r0, RPT)], shared.at[pl.ds(r0, RPT)])

        @pl.when(s == NS - 1)
        def _():
            pltpu.sync_copy(h_ref.at[pl.ds(15 * RPT, RPT_LAST)],
                            shared.at[pl.ds(15 * RPT, RPT_LAST)])

        plsc.subcore_barrier()

        def grp(u, carry):
            off = pl.multiple_of(s * EPT + u * Q, 8)
            pltpu.sync_copy(src.at[pl.ds(off, Q)], idx_s)
            pltpu.sync_copy(dst.at[pl.ds(off, Q)], idx_d)
            pltpu.async_copy(h_ref.at[idx_s], rows, gsem).wait()
            pltpu.sync_copy(rows, shared.at[idx_d], add=True)
            return carry

        lax.fori_loop(0, N_GRP, grp, 0)
        plsc.subcore_barrier()

        @pl.when(s < NS - 1)
        def _():
            pltpu.sync_copy(shared.at[pl.ds(r0, RPT)],
                            out.at[pl.ds(r0, RPT), pl.ds(co, G // 2)])

        @pl.when(s == NS - 1)
        def _():
            pltpu.sync_copy(shared.at[pl.ds(15 * RPT, RPT_LAST)],
                            out.at[pl.ds(15 * RPT, RPT_LAST), pl.ds(co, G // 2)])

        plsc.subcore_barrier()

    @pl.when(c == 0)
    def _():
        halfpass(h0, srcA, dstA, 0)
        halfpass(h2, srcT, dstT, G)

    @pl.when(c == 1)
    def _():
        halfpass(h1, srcA, dstA, G // 2)
        halfpass(h3, srcT, dstT, G + G // 2)


def _sc_layer(h0, h1, h2, h3, srcA, dstA, srcT, dstT):
    return pl.kernel(
        _layer_body,
        out_type=jax.ShapeDtypeStruct((N, 2 * G), jnp.float32),
        mesh=_mesh,
        scratch_types=[
            pltpu.VMEM((Q,), jnp.int32),
            pltpu.VMEM((Q,), jnp.int32),
            pltpu.VMEM((Q, G // 2), jnp.float32),
            pltpu.VMEM_SHARED((N + 8, G // 2), jnp.float32),
            pltpu.SemaphoreType.DMA,
        ],
        compiler_params=_sc_params,
    )(h0, h1, h2, h3, srcA, dstA, srcT, dstT)


_B = 5000
_GRID = N // _B


def _relu(x):
    return jnp.maximum(x, 0.0)


def _dot(a, b):
    return jnp.dot(a, b, preferred_element_type=jnp.float32)


def _enc_body(ctx, tlog, mf, degA, degT,
              W1, b1, g1, be1, m1, v1, W2, b2, Wt, bt, mtok, gW_a, gW_t,
              fused_o, hA0, hA1, hT0, hT1):
    h = _dot(ctx[...], W1[...]) + b1[...]
    h = _relu((h - m1[...]) * lax.rsqrt(v1[...] + 1e-5) * g1[...] + be1[...])
    ctx_emb = _relu(_dot(h, W2[...]) + b2[...])
    mfv = mf[...]
    mt = tlog[...] * (1.0 - mfv) + mtok[...] * mfv
    tgt_emb = _relu(_dot(mt, Wt[...]) + bt[...])
    fused = jnp.concatenate([ctx_emb, tgt_emb], axis=-1)
    fused_o[...] = fused
    dinvA = lax.rsqrt(degA[...] + 1.0)
    dinvT = lax.rsqrt(degT[...] + 1.0)
    ha = dinvA * _dot(fused, gW_a[...])
    ht = dinvT * _dot(fused, gW_t[...])
    hA0[...] = ha[:, :G // 2]
    hA1[...] = ha[:, G // 2:]
    hT0[...] = ht[:, :G // 2]
    hT1[...] = ht[:, G // 2:]


def _mid_body(acc, degA, degT, b_a, b_t, W2a, W2t, hA0, hA1, hT0, hT1):
    dinvA = lax.rsqrt(degA[...] + 1.0)
    dinvT = lax.rsqrt(degT[...] + 1.0)
    accv = acc[...]
    hs = _relu(dinvA * accv[:, :G] + b_a[...])
    ht = _relu(dinvT * accv[:, G:] + b_t[...])
    t2 = dinvA * _dot(hs, W2a[...])
    u2 = dinvT * _dot(ht, W2t[...])
    hA0[...] = t2[:, :G // 2]
    hA1[...] = t2[:, G // 2:]
    hT0[...] = u2[:, :G // 2]
    hT1[...] = u2[:, G // 2:]


def _fin_body(acc, degA, degT, fused, b_a, b_t,
              Wp1, bp1, g2, be2, m2, v2, Wp2, bp2, Wp3, bp3, out):
    dinvA = lax.rsqrt(degA[...] + 1.0)
    dinvT = lax.rsqrt(degT[...] + 1.0)
    accv = acc[...]
    hs2 = dinvA * accv[:, :G] + b_a[...]
    ht2 = dinvT * accv[:, G:] + b_t[...]
    combined = jnp.concatenate([hs2 + ht2, fused[...]], axis=-1)
    o = _dot(combined, Wp1[...]) + bp1[...]
    o = _relu((o - m2[...]) * lax.rsqrt(v2[...] + 1e-5) * g2[...] + be2[...])
    o = _relu(_dot(o, Wp2[...]) + bp2[...])
    out[...] = _dot(o, Wp3[...]) + bp3[...]


def _row_spec(w):
    return pl.BlockSpec((_B, w), lambda i: (i, 0))


def _full_spec(a, b):
    return pl.BlockSpec((a, b), lambda i: (0, 0))


def _tc_call(body, in_specs, out_specs, out_shapes):
    return pl.pallas_call(
        body,
        grid=(_GRID,),
        in_specs=in_specs,
        out_specs=out_specs,
        out_shape=out_shapes,
        compiler_params=pltpu.CompilerParams(
            dimension_semantics=("arbitrary",)),
    )


def kernel(context, target_log, mask, adj_edge_index, transit_edge_index,
           W1, b1, bn1_g, bn1_b, bn1_m, bn1_v, W2, b2, Wt, bt, mask_token,
           gcn1_W, gcn1_b, gcn2_W, gcn2_b, tg1_W, tg1_b, tg2_W, tg2_b,
           graph_alpha, Wp1, bp1, bn2_g, bn2_b, bn2_m, bn2_v, Wp2, bp2,
           Wp3, bp3):
    mf = mask.astype(jnp.float32)
    pad_src = jnp.zeros((E_PAD - E,), jnp.int32)
    pad_dst = jnp.full((E_PAD - E,), PAD_DST, jnp.int32)
    srcA = jnp.concatenate([adj_edge_index[0], pad_src])
    dstA = jnp.concatenate([adj_edge_index[1], pad_dst])
    srcT = jnp.concatenate([transit_edge_index[0], pad_src])
    dstT = jnp.concatenate([transit_edge_index[1], pad_dst])
    alpha = jax.nn.sigmoid(graph_alpha)
    gcn2_Ws = gcn2_W * alpha
    gcn2_bs = (gcn2_b * alpha).reshape(1, G)
    tg2_Ws = tg2_W * (1.0 - alpha)
    tg2_bs = (tg2_b * (1.0 - alpha)).reshape(1, G)
    row = lambda v: v.reshape(1, -1)

    degA, degT = _sc_degrees(dstA, dstT)
    degA = degA[:N].reshape(N, 1)
    degT = degT[:N].reshape(N, 1)

    enc = _tc_call(
        _enc_body,
        in_specs=[
            _row_spec(CTX), _row_spec(TGT), _row_spec(TGT),
            _row_spec(1), _row_spec(1),
            _full_spec(CTX, H), _full_spec(1, H),
            _full_spec(1, H), _full_spec(1, H), _full_spec(1, H), _full_spec(1, H),
            _full_spec(H, H), _full_spec(1, H),
            _full_spec(TGT, H // 2), _full_spec(1, H // 2), _full_spec(1, TGT),
            _full_spec(FUSION, G), _full_spec(FUSION, G),
        ],
        out_specs=[_row_spec(FUSION)] + [_row_spec(G // 2)] * 4,
        out_shapes=[jax.ShapeDtypeStruct((N, FUSION), jnp.float32)] +
                   [jax.ShapeDtypeStruct((N, G // 2), jnp.float32)] * 4,
    )
    fused, hA0, hA1, hT0, hT1 = enc(
        context, target_log, mf, degA, degT,
        W1, row(b1), row(bn1_g), row(bn1_b), row(bn1_m), row(bn1_v),
        W2, row(b2), Wt, row(bt), mask_token, gcn1_W, tg1_W)

    acc1 = _sc_layer(hA0, hA1, hT0, hT1, srcA, dstA, srcT, dstT)

    mid = _tc_call(
        _mid_body,
        in_specs=[_row_spec(2 * G)] + [_row_spec(1)] * 2 +
                 [_full_spec(1, G)] * 2 + [_full_spec(G, G)] * 2,
        out_specs=[_row_spec(G // 2)] * 4,
        out_shapes=[jax.ShapeDtypeStruct((N, G // 2), jnp.float32)] * 4,
    )
    hA20, hA21, hT20, hT21 = mid(
        acc1, degA, degT, row(gcn1_b), row(tg1_b), gcn2_Ws, tg2_Ws)

    acc2 = _sc_layer(hA20, hA21, hT20, hT21, srcA, dstA, srcT, dstT)

    fin = _tc_call(
        _fin_body,
        in_specs=[_row_spec(2 * G)] + [_row_spec(1)] * 2 +
                 [_row_spec(FUSION)] + [_full_spec(1, G)] * 2 +
                 [_full_spec(G + FUSION, H), _full_spec(1, H)] +
                 [_full_spec(1, H)] * 4 +
                 [_full_spec(H, H // 2), _full_spec(1, H // 2),
                  _full_spec(H // 2, TGT), _full_spec(1, TGT)],
        out_specs=[_row_spec(TGT)],
        out_shapes=[jax.ShapeDtypeStruct((N, TGT), jnp.float32)],
    )
    (out,) = fin(
        acc2, degA, degT, fused,
        gcn2_bs, tg2_bs, Wp1, row(bp1), row(bn2_g), row(bn2_b),
        row(bn2_m), row(bn2_v), Wp2, row(bp2), Wp3, row(bp3))
    return out


# speedup vs baseline: 1.0685x; 1.0435x over previous
import functools

import jax
import jax.numpy as jnp
from jax import lax
from jax.experimental import pallas as pl
from jax.experimental.pallas import tpu as pltpu
from jax.experimental.pallas import tpu_sc as plsc

N = 50000
E = 800000
CTX = 128
TGT = 32
H = 64
G = 64
FUSION = H + H // 2

NC = 2
NS = 16
EPT = E // NS
Q = 224
N_GRP = EPT // Q
QT = EPT - N_GRP * Q
RPT = 3128
RPT_LAST = N - 15 * RPT
DEG_PAD = N + 48
DRPT = DEG_PAD // NS

_mesh = plsc.VectorSubcoreMesh(core_axis_name="c", subcore_axis_name="s")
_sc_params = pltpu.CompilerParams(use_tc_tiling_on_sc=False)


def _deg_body(eiA, eiT, zf, onesf, degA, degT, idx2, ones_v, shared, sem):
    c = lax.axis_index("c")
    s = lax.axis_index("s")
    pltpu.sync_copy(onesf, ones_v)

    def work(ei_ref, out_ref):
        r0 = pl.multiple_of(s * DRPT, 8)
        pltpu.sync_copy(zf, shared.at[pl.ds(r0, DRPT)])
        c0 = pl.multiple_of(s * EPT, 8)
        pltpu.sync_copy(ei_ref.at[1].at[pl.ds(c0, EPT)], idx2)
        plsc.subcore_barrier()
        pltpu.sync_copy(ones_v, shared.at[idx2], add=True)
        plsc.subcore_barrier()
        pltpu.sync_copy(shared.at[pl.ds(r0, DRPT)], out_ref.at[pl.ds(r0, DRPT)])

    @pl.when(c == 0)
    def _():
        work(eiA, degA)

    @pl.when(c == 1)
    def _():
        work(eiT, degT)


def _sc_degrees(eiA, eiT):
    zf = jnp.zeros((DRPT,), jnp.float32)
    onesf = jnp.ones((EPT,), jnp.float32)
    return pl.kernel(
        _deg_body,
        out_type=(jax.ShapeDtypeStruct((DEG_PAD,), jnp.float32),
                  jax.ShapeDtypeStruct((DEG_PAD,), jnp.float32)),
        mesh=_mesh,
        scratch_types=[
            pltpu.VMEM((EPT,), jnp.int32),
            pltpu.VMEM((EPT,), jnp.float32),
            pltpu.VMEM_SHARED((DEG_PAD,), jnp.float32),
            pltpu.SemaphoreType.DMA,
        ],
        compiler_params=_sc_params,
    )(eiA, eiT, zf, onesf)


def _layer_body(h0, h1, h2, h3, eiA, eiT, out,
                idx_s, idx_d, idx_st, idx_dt, rows, shared, gsem):
    c = lax.axis_index("c")
    s = lax.axis_index("s")

    def halfpass(h_ref, ei_ref, co):
        src = ei_ref.at[0]
        dst = ei_ref.at[1]
        r0 = pl.multiple_of(s * RPT, 8)

        @pl.when(s < NS - 1)
        def _():
            pltpu.sync_copy(h_ref.at[pl.ds(r0, RPT)], shared.at[pl.ds(r0, RPT)])

        @pl.when(s == NS - 1)
        def _():
            pltpu.sync_copy(h_ref.at[pl.ds(15 * RPT, RPT_LAST)],
                            shared.at[pl.ds(15 * RPT, RPT_LAST)])

        plsc.subcore_barrier()

        def grp(u, carry):
            off = pl.multiple_of(s * EPT + u * Q, 8)
            pltpu.sync_copy(src.at[pl.ds(off, Q)], idx_s)
            pltpu.sync_copy(dst.at[pl.ds(off, Q)], idx_d)
            pltpu.async_copy(h_ref.at[idx_s], rows, gsem).wait()
            pltpu.sync_copy(rows, shared.at[idx_d], add=True)
            return carry

        lax.fori_loop(0, N_GRP, grp, 0)
        offt = pl.multiple_of(s * EPT + N_GRP * Q, 8)
        pltpu.sync_copy(src.at[pl.ds(offt, QT)], idx_st)
        pltpu.sync_copy(dst.at[pl.ds(offt, QT)], idx_dt)
        rows_t = rows.at[pl.ds(0, QT)]
        pltpu.async_copy(h_ref.at[idx_st], rows_t, gsem).wait()
        pltpu.sync_copy(rows_t, shared.at[idx_dt], add=True)
        plsc.subcore_barrier()

        @pl.when(s < NS - 1)
        def _():
            pltpu.sync_copy(shared.at[pl.ds(r0, RPT)],
                            out.at[pl.ds(r0, RPT), pl.ds(co, G // 2)])

        @pl.when(s == NS - 1)
        def _():
            pltpu.sync_copy(shared.at[pl.ds(15 * RPT, RPT_LAST)],
                            out.at[pl.ds(15 * RPT, RPT_LAST), pl.ds(co, G // 2)])

        plsc.subcore_barrier()

    @pl.when(c == 0)
    def _():
        halfpass(h0, eiA, 0)
        halfpass(h2, eiT, G)

    @pl.when(c == 1)
    def _():
        halfpass(h1, eiA, G // 2)
        halfpass(h3, eiT, G + G // 2)


def _sc_layer(h0, h1, h2, h3, eiA, eiT):
    return pl.kernel(
        _layer_body,
        out_type=jax.ShapeDtypeStruct((N, 2 * G), jnp.float32),
        mesh=_mesh,
        scratch_types=[
            pltpu.VMEM((Q,), jnp.int32),
            pltpu.VMEM((Q,), jnp.int32),
            pltpu.VMEM((QT,), jnp.int32),
            pltpu.VMEM((QT,), jnp.int32),
            pltpu.VMEM((Q, G // 2), jnp.float32),
            pltpu.VMEM_SHARED((N + 8, G // 2), jnp.float32),
            pltpu.SemaphoreType.DMA,
        ],
        compiler_params=_sc_params,
    )(h0, h1, h2, h3, eiA, eiT)


_B = 5000
_GRID = N // _B


def _relu(x):
    return jnp.maximum(x, 0.0)


def _dot(a, b):
    return jnp.dot(a, b, preferred_element_type=jnp.float32)


def _enc_body(ctx, tlog, mf, degA, degT,
              W1, b1, g1, be1, m1, v1, W2, b2, Wt, bt, mtok, gW_a, gW_t,
              fused_o, hA0, hA1, hT0, hT1):
    h = _dot(ctx[...], W1[...]) + b1[...]
    h = _relu((h - m1[...]) * lax.rsqrt(v1[...] + 1e-5) * g1[...] + be1[...])
    ctx_emb = _relu(_dot(h, W2[...]) + b2[...])
    mfv = mf[...]
    mt = tlog[...] * (1.0 - mfv) + mtok[...] * mfv
    tgt_emb = _relu(_dot(mt, Wt[...]) + bt[...])
    fused = jnp.concatenate([ctx_emb, tgt_emb], axis=-1)
    fused_o[...] = fused
    dinvA = lax.rsqrt(degA[...] + 1.0)
    dinvT = lax.rsqrt(degT[...] + 1.0)
    ha = dinvA * _dot(fused, gW_a[...])
    ht = dinvT * _dot(fused, gW_t[...])
    hA0[...] = ha[:, :G // 2]
    hA1[...] = ha[:, G // 2:]
    hT0[...] = ht[:, :G // 2]
    hT1[...] = ht[:, G // 2:]


def _mid_body(acc, degA, degT, b_a, b_t, W2a, W2t, hA0, hA1, hT0, hT1):
    dinvA = lax.rsqrt(degA[...] + 1.0)
    dinvT = lax.rsqrt(degT[...] + 1.0)
    accv = acc[...]
    hs = _relu(dinvA * accv[:, :G] + b_a[...])
    ht = _relu(dinvT * accv[:, G:] + b_t[...])
    t2 = dinvA * _dot(hs, W2a[...])
    u2 = dinvT * _dot(ht, W2t[...])
    hA0[...] = t2[:, :G // 2]
    hA1[...] = t2[:, G // 2:]
    hT0[...] = u2[:, :G // 2]
    hT1[...] = u2[:, G // 2:]


def _fin_body(acc, degA, degT, fused, b_a, b_t,
              Wp1, bp1, g2, be2, m2, v2, Wp2, bp2, Wp3, bp3, out):
    dinvA = lax.rsqrt(degA[...] + 1.0)
    dinvT = lax.rsqrt(degT[...] + 1.0)
    accv = acc[...]
    hs2 = dinvA * accv[:, :G] + b_a[...]
    ht2 = dinvT * accv[:, G:] + b_t[...]
    combined = jnp.concatenate([hs2 + ht2, fused[...]], axis=-1)
    o = _dot(combined, Wp1[...]) + bp1[...]
    o = _relu((o - m2[...]) * lax.rsqrt(v2[...] + 1e-5) * g2[...] + be2[...])
    o = _relu(_dot(o, Wp2[...]) + bp2[...])
    out[...] = _dot(o, Wp3[...]) + bp3[...]


def _row_spec(w):
    return pl.BlockSpec((_B, w), lambda i: (i, 0))


def _full_spec(a, b):
    return pl.BlockSpec((a, b), lambda i: (0, 0))


def _tc_call(body, in_specs, out_specs, out_shapes):
    return pl.pallas_call(
        body,
        grid=(_GRID,),
        in_specs=in_specs,
        out_specs=out_specs,
        out_shape=out_shapes,
        compiler_params=pltpu.CompilerParams(
            dimension_semantics=("arbitrary",)),
    )


def kernel(context, target_log, mask, adj_edge_index, transit_edge_index,
           W1, b1, bn1_g, bn1_b, bn1_m, bn1_v, W2, b2, Wt, bt, mask_token,
           gcn1_W, gcn1_b, gcn2_W, gcn2_b, tg1_W, tg1_b, tg2_W, tg2_b,
           graph_alpha, Wp1, bp1, bn2_g, bn2_b, bn2_m, bn2_v, Wp2, bp2,
           Wp3, bp3):
    mf = mask.astype(jnp.float32)
    alpha = jax.nn.sigmoid(graph_alpha)
    gcn2_Ws = gcn2_W * alpha
    gcn2_bs = (gcn2_b * alpha).reshape(1, G)
    tg2_Ws = tg2_W * (1.0 - alpha)
    tg2_bs = (tg2_b * (1.0 - alpha)).reshape(1, G)
    row = lambda v: v.reshape(1, -1)

    degA, degT = _sc_degrees(adj_edge_index, transit_edge_index)
    degA = degA[:N].reshape(N, 1)
    degT = degT[:N].reshape(N, 1)

    enc = _tc_call(
        _enc_body,
        in_specs=[
            _row_spec(CTX), _row_spec(TGT), _row_spec(TGT),
            _row_spec(1), _row_spec(1),
            _full_spec(CTX, H), _full_spec(1, H),
            _full_spec(1, H), _full_spec(1, H), _full_spec(1, H), _full_spec(1, H),
            _full_spec(H, H), _full_spec(1, H),
            _full_spec(TGT, H // 2), _full_spec(1, H // 2), _full_spec(1, TGT),
            _full_spec(FUSION, G), _full_spec(FUSION, G),
        ],
        out_specs=[_row_spec(FUSION)] + [_row_spec(G // 2)] * 4,
        out_shapes=[jax.ShapeDtypeStruct((N, FUSION), jnp.float32)] +
                   [jax.ShapeDtypeStruct((N, G // 2), jnp.float32)] * 4,
    )
    fused, hA0, hA1, hT0, hT1 = enc(
        context, target_log, mf, degA, degT,
        W1, row(b1), row(bn1_g), row(bn1_b), row(bn1_m), row(bn1_v),
        W2, row(b2), Wt, row(bt), mask_token, gcn1_W, tg1_W)

    acc1 = _sc_layer(hA0, hA1, hT0, hT1, adj_edge_index, transit_edge_index)

    mid = _tc_call(
        _mid_body,
        in_specs=[_row_spec(2 * G)] + [_row_spec(1)] * 2 +
                 [_full_spec(1, G)] * 2 + [_full_spec(G, G)] * 2,
        out_specs=[_row_spec(G // 2)] * 4,
        out_shapes=[jax.ShapeDtypeStruct((N, G // 2), jnp.float32)] * 4,
    )
    hA20, hA21, hT20, hT21 = mid(
        acc1, degA, degT, row(gcn1_b), row(tg1_b), gcn2_Ws, tg2_Ws)

    acc2 = _sc_layer(hA20, hA21, hT20, hT21, adj_edge_index,
                     transit_edge_index)

    fin = _tc_call(
        _fin_body,
        in_specs=[_row_spec(2 * G)] + [_row_spec(1)] * 2 +
                 [_row_spec(FUSION)] + [_full_spec(1, G)] * 2 +
                 [_full_spec(G + FUSION, H), _full_spec(1, H)] +
                 [_full_spec(1, H)] * 4 +
                 [_full_spec(H, H // 2), _full_spec(1, H // 2),
                  _full_spec(H // 2, TGT), _full_spec(1, TGT)],
        out_specs=[_row_spec(TGT)],
        out_shapes=[jax.ShapeDtypeStruct((N, TGT), jnp.float32)],
    )
    (out,) = fin(
        acc2, degA, degT, fused,
        gcn2_bs, tg2_bs, Wp1, row(bp1), row(bn2_g), row(bn2_b),
        row(bn2_m), row(bn2_v), Wp2, row(bp2), Wp3, row(bp3))
    return out
